# baseline (speedup 1.0000x reference)
import functools

import jax
import jax.numpy as jnp
from jax import lax
from jax.experimental import pallas as pl
from jax.experimental.pallas import tpu as pltpu
from jax.experimental.pallas import tpu_sc as plsc

BATCH = 32
SEQ = 256
TOK = BATCH * SEQ
D = 1024
EPS = 1e-3
L = 16
NC = 2
NS = 16
NW = NC * NS
TPW = TOK // NW
C = 32
NCHUNK = TPW // C


def _ln_kernel(x0_h, x1_h, emb0_h, emb1_h, pos_h, a2_h, b2_h,
               out_h, tim_h,
               idx0_v, idx1_v, rows0_v, rows1_v, pos_v, a2_v, b2_v,
               sem0, sem1):
    cid = lax.axis_index("c")
    sid = lax.axis_index("s")
    wid = sid * NC + cid
    base = wid * TPW

    pltpu.sync_copy(a2_h, a2_v)
    pltpu.sync_copy(b2_h, b2_v)

    def chunk_body(ci, _):
        tok0 = base + ci * C
        pltpu.sync_copy(x0_h.at[pl.ds(tok0, C)], idx0_v)
        pltpu.sync_copy(x1_h.at[pl.ds(tok0, C)], idx1_v)
        cp0 = pltpu.async_copy(emb0_h.at[idx0_v], rows0_v, sem0)
        cp1 = pltpu.async_copy(emb1_h.at[idx1_v], rows1_v, sem1)
        pltpu.sync_copy(pos_h.at[pl.ds(ci * C, C)], pos_v)
        cp0.wait()
        cp1.wait()

        def tok_body(t, _):
            def j_body(j, carry):
                s, sq = carry
                z = (rows0_v[t, pl.ds(j * L, L)]
                     + rows1_v[t, pl.ds(j * L, L)]
                     + pos_v[t, pl.ds(j * L, L)])
                rows0_v[t, pl.ds(j * L, L)] = z
                return (s + z, sq + z * z)

            zero = jnp.zeros((L,), jnp.float32)
            s, sq = lax.fori_loop(0, D // L, j_body, (zero, zero),
                                  unroll=4)
            dnums = lax.GatherDimensionNumbers(
                offset_dims=(), collapsed_slice_dims=(0,),
                start_index_map=(0,))
            shuf = functools.partial(
                lax.gather, dimension_numbers=dnums, slice_sizes=(1,),
                mode=lax.GatherScatterMode.PROMISE_IN_BOUNDS)
            lane = lax.iota(jnp.int32, L)
            for step in (8, 4, 2, 1):
                perm = (lane ^ step).reshape(L, 1)
                s = s + shuf(s, perm)
                sq = sq + shuf(sq, perm)
            ssum, ssq = s, sq
            muv = ssum * (1.0 / D)
            var = (ssq - ssum * muv) * (1.0 / (D - 1))
            yi = (jnp.int32(0x5F3759DF)
                  - (lax.bitcast_convert_type(var, jnp.int32) >> 1))
            y = lax.bitcast_convert_type(yi, jnp.float32)
            half = var * 0.5
            for _ in range(4):
                y = y * (1.5 - half * y * y)
            sigma = var * y
            scale = 1.0 / (sigma + EPS)

            def j2_body(j, carry):
                z = rows0_v[t, pl.ds(j * L, L)]
                rows0_v[t, pl.ds(j * L, L)] = (
                    (z - muv) * scale * a2_v[pl.ds(j * L, L)]
                    + b2_v[pl.ds(j * L, L)])
                return carry

            lax.fori_loop(0, D // L, j2_body, 0, unroll=4)
            return 0

        lax.fori_loop(0, C, tok_body, 0)
        pltpu.sync_copy(rows0_v, out_h.at[pl.ds(tok0, C)])
        pltpu.sync_copy(pos_v, tim_h.at[pl.ds(tok0, C)])
        return 0

    lax.fori_loop(0, NCHUNK, chunk_body, 0)


def kernel(x0, x1, emb0, emb1, position_table, a_2, b_2):
    mesh = plsc.VectorSubcoreMesh(core_axis_name="c", subcore_axis_name="s")
    f = pl.kernel(
        _ln_kernel,
        out_type=(
            jax.ShapeDtypeStruct((TOK, D), jnp.float32),
            jax.ShapeDtypeStruct((TOK, D), jnp.float32),
        ),
        mesh=mesh,
        scratch_types=[
            pltpu.VMEM((C,), jnp.int32),
            pltpu.VMEM((C,), jnp.int32),
            pltpu.VMEM((C, D), jnp.float32),
            pltpu.VMEM((C, D), jnp.float32),
            pltpu.VMEM((C, D), jnp.float32),
            pltpu.VMEM((D,), jnp.float32),
            pltpu.VMEM((D,), jnp.float32),
            pltpu.SemaphoreType.DMA,
            pltpu.SemaphoreType.DMA,
        ],
    )
    return f(x0.astype(jnp.int32), x1.astype(jnp.int32),
             emb0, emb1, position_table, a_2, b_2)
